# baseline (speedup 1.0000x reference)
import functools

import jax
import jax.numpy as jnp
from jax import lax
from jax.experimental import pallas as pl
from jax.experimental.pallas import tpu as pltpu
from jax.experimental.pallas import tpu_sc as plsc

_N = 100000
_E = 1600000
_B = 2048
_AC = 16
_CC = 6
_BC = 5

_LAA = 8192
_LAB = 6400
_NP = 131072
_NROWS = _NP // 128
_EROWS_PAD = 12544
_EP = _EROWS_PAD * 128

_NC = 2
_NS = 16
_NW = _NC * _NS

_BROWS_W = _EROWS_PAD // _NW
_BBLK = 56
_BNBLK = _BROWS_W // _BBLK
_AROWS_W = _NROWS // _NW
_TILE_N = _NP // _NS


def _atom_body(pc_ref, tc_ref, pa_ref, ta_ref, pch_ref, tch_ref,
               regr_ref, ace_ref, cce_ref):
    blk = pl.program_id(0)
    lanes = lax.broadcasted_iota(jnp.int32, (1, _LAA), 1) + blk * _LAA
    mask = lanes < _N

    d = pc_ref[...] - tc_ref[...]
    regr = jnp.sum(d * d, axis=0, keepdims=True) * (1.0 / 3.0)
    regr_ref[...] = jnp.where(mask, regr, 0.0)[None]

    pa = pa_ref[...]
    ta = ta_ref[...]
    m = jnp.max(pa, axis=0, keepdims=True)
    lse = jnp.log(jnp.sum(jnp.exp(pa - m), axis=0, keepdims=True)) + m
    onehot = lax.broadcasted_iota(jnp.int32, (_AC, _LAA), 0) == ta
    tgt = jnp.sum(jnp.where(onehot, pa, 0.0), axis=0, keepdims=True)
    ace_ref[...] = jnp.where(mask, lse - tgt, 0.0)[None]

    pch = pch_ref[...]
    tch = tch_ref[...]
    m2 = jnp.max(pch, axis=0, keepdims=True)
    lse2 = jnp.log(jnp.sum(jnp.exp(pch - m2), axis=0, keepdims=True)) + m2
    onehot2 = lax.broadcasted_iota(jnp.int32, (_CC, _LAA), 0) == tch
    tgt2 = jnp.sum(jnp.where(onehot2, pch, 0.0), axis=0, keepdims=True)
    cce_ref[...] = jnp.where(mask, lse2 - tgt2, 0.0)[None]


_ANBLK = 13


def _atom_values(pc_t, tc_t, pa_t, ta2, pch_t, tch2):
    out_sds = jax.ShapeDtypeStruct((_ANBLK, 1, _LAA), jnp.float32)
    lane = lambda r: pl.BlockSpec((r, _LAA), lambda i: (0, i))
    return pl.pallas_call(
        _atom_body,
        grid=(_ANBLK,),
        in_specs=[lane(3), lane(3), lane(_AC), lane(1), lane(_CC), lane(1)],
        out_specs=[pl.BlockSpec((1, 1, _LAA), lambda i: (i, 0, 0))] * 3,
        out_shape=[out_sds, out_sds, out_sds],
    )(pc_t, tc_t, pa_t, ta2, pch_t, tch2)


_BCE_ROWS = 896


_EROWS = _E // 128


def _bond_body(p0_ref, p1_ref, p2_ref, p3_ref, p4_ref, tb_ref, idx_ref,
               ce_ref, idx2_ref):
    i = pl.program_id(0)
    ps = [p0_ref[0], p1_ref[0], p2_ref[0], p3_ref[0], p4_ref[0]]
    tb = tb_ref[...]
    m = jnp.maximum(jnp.maximum(jnp.maximum(ps[0], ps[1]),
                                jnp.maximum(ps[2], ps[3])), ps[4])
    ssum = jnp.exp(ps[0] - m)
    for k in range(1, _BC):
        ssum = ssum + jnp.exp(ps[k] - m)
    lse = jnp.log(ssum) + m
    tgt = jnp.where(tb == 0, ps[0], 0.0)
    for k in range(1, _BC):
        tgt = jnp.where(tb == k, ps[k], tgt)
    ce_ref[...] = lse - tgt

    rows = lax.broadcasted_iota(jnp.int32, (_BCE_ROWS, 128), 0) + i * _BCE_ROWS
    idx2_ref[...] = jnp.where(rows < _EROWS, idx_ref[...], _N)


def _bond_ce(pb_tp, tb2, idxr):
    nblk = _EROWS_PAD // _BCE_ROWS
    pspecs = [pl.BlockSpec((1, _BCE_ROWS, 128),
                           lambda i, k=k: (k, i, 0)) for k in range(_BC)]
    rspec = pl.BlockSpec((_BCE_ROWS, 128), lambda i: (i, 0))
    sds = jax.ShapeDtypeStruct((_EROWS_PAD, 128), jnp.float32)
    sdsi = jax.ShapeDtypeStruct((_EROWS_PAD, 128), jnp.int32)
    return pl.pallas_call(
        _bond_body,
        grid=(nblk,),
        in_specs=pspecs + [rspec, rspec],
        out_specs=[rspec, rspec],
        out_shape=[sds, sdsi],
    )(pb_tp, pb_tp, pb_tp, pb_tp, pb_tp, tb2, idxr)


def _zero_vmem(ref, nwords):
    z = jnp.zeros((16,), jnp.float32)

    def body(i, _):
        ref[pl.ds(i * 16, 16)] = z
        return 0

    lax.fori_loop(0, nwords // 16, body, 0)


def _bond_scatter_body(ce_hbm, idx_hbm, s0_hbm, s1_hbm, c0_hbm, c1_hbm,
                       idx_v, val_v, ones_v, zero_v, out_v2,
                       acc_s, cnt_s, sem):
    ci = lax.axis_index("c")
    si = lax.axis_index("s")
    wid = si * _NC + ci

    one = jnp.ones((16,), jnp.float32)
    for v in range(8):
        ones_v[pl.ds(v * 16, 16)] = one
    _zero_vmem(zero_v, _TILE_N)
    pltpu.sync_copy(zero_v, acc_s.at[pl.ds(si * _TILE_N, _TILE_N)])
    pltpu.sync_copy(zero_v, cnt_s.at[pl.ds(si * _TILE_N, _TILE_N)])
    plsc.subcore_barrier()

    def blk_body(bi, _):
        base = wid * _BROWS_W + bi * _BBLK
        pltpu.sync_copy(idx_hbm.at[pl.ds(base, _BBLK)], idx_v)
        pltpu.sync_copy(ce_hbm.at[pl.ds(base, _BBLK)], val_v)
        for g in range(0, _BBLK, 7):
            descs = []
            for j in range(g, g + 7):
                descs.append(pltpu.async_copy(
                    val_v.at[j], acc_s.at[idx_v.at[j]], sem, add=True))
                descs.append(pltpu.async_copy(
                    ones_v, cnt_s.at[idx_v.at[j]], sem, add=True))
            for d in descs:
                d.wait()
        return 0

    lax.fori_loop(0, _BNBLK, blk_body, 0)
    plsc.subcore_barrier()

    base = si * _TILE_N
    outs = [(s0_hbm, c0_hbm), (s1_hbm, c1_hbm)]
    for c in range(_NC):
        @pl.when(ci == c)
        def _():
            for src, dst in ((acc_s, outs[c][0]), (cnt_s, outs[c][1])):
                descs = [pltpu.async_copy(
                    src.at[pl.ds(base + r * 128, 128)], out_v2.at[r], sem)
                    for r in range(64)]
                for d in descs:
                    d.wait()
                pltpu.sync_copy(out_v2, dst.at[si])


def _bond_scatter(ce2, idx2):
    mesh = plsc.VectorSubcoreMesh(core_axis_name="c", subcore_axis_name="s",
                                  num_cores=_NC, num_subcores=_NS)
    sds = jax.ShapeDtypeStruct((_NS, 64, 128), jnp.float32)
    f = pl.kernel(
        _bond_scatter_body,
        out_type=[sds, sds, sds, sds],
        mesh=mesh,
        scratch_types=[
            pltpu.VMEM((_BBLK, 128), jnp.int32),
            pltpu.VMEM((_BBLK, 128), jnp.float32),
            pltpu.VMEM((128,), jnp.float32),
            pltpu.VMEM((_TILE_N,), jnp.float32),
            pltpu.VMEM((64, 128), jnp.float32),
            pltpu.VMEM_SHARED((_NP,), jnp.float32),
            pltpu.VMEM_SHARED((_NP,), jnp.float32),
            pltpu.SemaphoreType.DMA,
        ],
    )
    return f(ce2, idx2)


_SEG_L = 8192
_SEG_NBLK = 13


def _onehot_hi(bat):
    return jnp.where(
        lax.broadcasted_iota(jnp.int32, (128, _SEG_L), 0) == bat // 16,
        1.0, 0.0).astype(jnp.bfloat16)


def _lomask(bat):
    return lax.broadcasted_iota(jnp.int32, (16, _SEG_L), 0) == bat % 16


def _seg1_body(bat_ref, regr_ref, ace_ref, cce_ref, out_ref, macc):
    i = pl.program_id(0)
    bat = bat_ref[...]
    valid = (lax.broadcasted_iota(jnp.int32, (1, _SEG_L), 1) + i * _SEG_L) < _N
    onesrow = jnp.where(valid, 1.0, 0.0)
    lom = _lomask(bat)
    vals = [regr_ref[0], ace_ref[0], cce_ref[0], onesrow]
    vm = jnp.concatenate(
        [jnp.where(lom, v, 0.0) for v in vals], axis=0).astype(jnp.bfloat16)
    contrib = lax.dot_general(_onehot_hi(bat), vm, (((1,), (1,)), ((), ())),
                              preferred_element_type=jnp.float32)

    @pl.when(i == 0)
    def _():
        macc[...] = contrib

    @pl.when(i > 0)
    def _():
        macc[...] = macc[...] + contrib

    @pl.when(i == _SEG_NBLK - 1)
    def _():
        out_ref[...] = macc[...]


def _seg1(bat2, regr3, ace3, cce3):
    spec = pl.BlockSpec((1, 1, _SEG_L), lambda i: (i, 0, 0))
    bspec = pl.BlockSpec((1, _SEG_L), lambda i: (0, i))
    return pl.pallas_call(
        _seg1_body,
        grid=(_SEG_NBLK,),
        in_specs=[bspec] + [spec] * 3,
        out_specs=pl.BlockSpec((128, 64), lambda i: (0, 0)),
        out_shape=jax.ShapeDtypeStruct((128, 64), jnp.float32),
        scratch_shapes=[pltpu.VMEM((128, 64), jnp.float32)],
    )(bat2, regr3, ace3, cce3)


def _seg2_body(bat_ref, s0_ref, s1_ref, c0_ref, c1_ref, m1_ref, w_ref,
               out_ref, macc):
    i = pl.program_id(0)
    bat = bat_ref[...]
    flat = (lax.broadcasted_iota(jnp.int32, (64, 128), 0) * 128
            + lax.broadcasted_iota(jnp.int32, (64, 128), 1) + i * _SEG_L)
    sm = s0_ref[0] + s1_ref[0]
    cm = c0_ref[0] + c1_ref[0]
    b2 = jnp.where(flat < _N, (0.5 * sm) / jnp.maximum(cm, 1.0), 0.0)
    b = b2.reshape(1, _SEG_L)
    lom = _lomask(bat)
    vm = jnp.where(lom, b, 0.0).astype(jnp.bfloat16)
    contrib = lax.dot_general(_onehot_hi(bat), vm, (((1,), (1,)), ((), ())),
                              preferred_element_type=jnp.float32)

    @pl.when(i == 0)
    def _():
        macc[...] = contrib

    @pl.when(i > 0)
    def _():
        macc[...] = macc[...] + contrib

    @pl.when(i == _SEG_NBLK - 1)
    def _():
        m1 = m1_ref[...]
        cnt = jnp.maximum(m1[:, 48:64], 1.0)
        w = w_ref[...]
        cols = []
        for k in range(3):
            mk = m1[:, k * 16:(k + 1) * 16] / cnt
            mk = jnp.where(jnp.isnan(mk), 0.0, mk * w)
            cols.append(jnp.sum(mk, axis=1, keepdims=True))
        mb = macc[...] / cnt
        mb = jnp.where(jnp.isnan(mb), 0.0, mb * w)
        cols.append(jnp.sum(mb, axis=1, keepdims=True))
        c4 = jnp.concatenate([cols[0], cols[1], cols[2], cols[3]], axis=1)
        out_ref[...] = jnp.sum(c4, axis=0, keepdims=True)


def _seg2(bat2, s03, s13, c03, c13, m1, w128):
    spec = pl.BlockSpec((1, 64, 128), lambda i: (i, 0, 0))
    bspec = pl.BlockSpec((1, _SEG_L), lambda i: (0, i))
    return pl.pallas_call(
        _seg2_body,
        grid=(_SEG_NBLK,),
        in_specs=[bspec] + [spec] * 4
        + [pl.BlockSpec((128, 64), lambda i: (0, 0)),
           pl.BlockSpec((128, 16), lambda i: (0, 0))],
        out_specs=pl.BlockSpec((1, 4), lambda i: (0, 0)),
        out_shape=jax.ShapeDtypeStruct((1, 4), jnp.float32),
        scratch_shapes=[pltpu.VMEM((128, 16), jnp.float32)],
    )(bat2, s03, s13, c03, c13, m1, w128)


@jax.jit
def kernel(pred_coords, true_coords, pred_atoms, true_atoms, pred_charges,
           true_charges, pred_bonds, true_bonds, batch,
           bond_aggregation_index, weights):
    idxr = bond_aggregation_index.reshape(_EROWS, 128)
    tb2 = true_bonds.reshape(_EROWS, 128)
    pb_tp = jnp.pad(pred_bonds, ((0, _EP - _E), (0, 0))).T.reshape(
        _BC, _EROWS_PAD, 128)
    ce, idx2 = _bond_ce(pb_tp, tb2, idxr)

    regr, ace, cce = _atom_values(
        pred_coords.T, true_coords.T, pred_atoms.T,
        true_atoms.reshape(1, _N), pred_charges.T,
        true_charges.reshape(1, _N))
    bat2 = batch.reshape(1, _N)
    m1 = _seg1(bat2, regr, ace, cce)

    s0, s1, c0, c1 = _bond_scatter(ce, idx2)
    out = _seg2(bat2, s0, s1, c0, c1, m1, weights.reshape(128, 16))
    return out.reshape(4)


# speedup vs baseline: 1.0077x; 1.0077x over previous
import functools

import jax
import jax.numpy as jnp
from jax import lax
from jax.experimental import pallas as pl
from jax.experimental.pallas import tpu as pltpu
from jax.experimental.pallas import tpu_sc as plsc

_N = 100000
_E = 1600000
_B = 2048
_AC = 16
_CC = 6
_BC = 5

_LAA = 8192
_LAB = 6400
_NP = 131072
_NROWS = _NP // 128
_EROWS_PAD = 12544
_EP = _EROWS_PAD * 128

_NC = 2
_NS = 16
_NW = _NC * _NS

_BROWS_W = _EROWS_PAD // _NW
_BBLK = 56
_BNBLK = _BROWS_W // _BBLK
_AROWS_W = _NROWS // _NW
_TILE_N = _NP // _NS


def _atom_body(pc_ref, tc_ref, pa_ref, ta_ref, pch_ref, tch_ref,
               regr_ref, ace_ref, cce_ref):
    blk = pl.program_id(0)
    lanes = lax.broadcasted_iota(jnp.int32, (1, _LAA), 1) + blk * _LAA
    mask = lanes < _N

    d = pc_ref[...] - tc_ref[...]
    regr = jnp.sum(d * d, axis=0, keepdims=True) * (1.0 / 3.0)
    regr_ref[...] = jnp.where(mask, regr, 0.0)[None]

    pa = pa_ref[...]
    ta = ta_ref[...]
    m = jnp.max(pa, axis=0, keepdims=True)
    lse = jnp.log(jnp.sum(jnp.exp(pa - m), axis=0, keepdims=True)) + m
    onehot = lax.broadcasted_iota(jnp.int32, (_AC, _LAA), 0) == ta
    tgt = jnp.sum(jnp.where(onehot, pa, 0.0), axis=0, keepdims=True)
    ace_ref[...] = jnp.where(mask, lse - tgt, 0.0)[None]

    pch = pch_ref[...]
    tch = tch_ref[...]
    m2 = jnp.max(pch, axis=0, keepdims=True)
    lse2 = jnp.log(jnp.sum(jnp.exp(pch - m2), axis=0, keepdims=True)) + m2
    onehot2 = lax.broadcasted_iota(jnp.int32, (_CC, _LAA), 0) == tch
    tgt2 = jnp.sum(jnp.where(onehot2, pch, 0.0), axis=0, keepdims=True)
    cce_ref[...] = jnp.where(mask, lse2 - tgt2, 0.0)[None]


_ANBLK = 13


def _atom_values(pc_t, tc_t, pa_t, ta2, pch_t, tch2):
    out_sds = jax.ShapeDtypeStruct((_ANBLK, 1, _LAA), jnp.float32)
    lane = lambda r: pl.BlockSpec((r, _LAA), lambda i: (0, i))
    return pl.pallas_call(
        _atom_body,
        grid=(_ANBLK,),
        in_specs=[lane(3), lane(3), lane(_AC), lane(1), lane(_CC), lane(1)],
        out_specs=[pl.BlockSpec((1, 1, _LAA), lambda i: (i, 0, 0))] * 3,
        out_shape=[out_sds, out_sds, out_sds],
    )(pc_t, tc_t, pa_t, ta2, pch_t, tch2)


_BCE_ROWS = 896


_EROWS = _E // 128


def _bond_body(p0_ref, p1_ref, p2_ref, p3_ref, p4_ref, tb_ref, idx_ref,
               ce_ref, idx2_ref):
    i = pl.program_id(0)
    ps = [p0_ref[0], p1_ref[0], p2_ref[0], p3_ref[0], p4_ref[0]]
    tb = tb_ref[...]
    m = jnp.maximum(jnp.maximum(jnp.maximum(ps[0], ps[1]),
                                jnp.maximum(ps[2], ps[3])), ps[4])
    ssum = jnp.exp(ps[0] - m)
    for k in range(1, _BC):
        ssum = ssum + jnp.exp(ps[k] - m)
    lse = jnp.log(ssum) + m
    tgt = jnp.where(tb == 0, ps[0], 0.0)
    for k in range(1, _BC):
        tgt = jnp.where(tb == k, ps[k], tgt)
    ce_ref[...] = lse - tgt

    rows = lax.broadcasted_iota(jnp.int32, (_BCE_ROWS, 128), 0) + i * _BCE_ROWS
    idx2_ref[...] = jnp.where(rows < _EROWS, idx_ref[...], _N)


def _bond_ce(pb_tp, tb2, idxr):
    nblk = _EROWS_PAD // _BCE_ROWS
    pspecs = [pl.BlockSpec((1, _BCE_ROWS, 128),
                           lambda i, k=k: (k, i, 0)) for k in range(_BC)]
    rspec = pl.BlockSpec((_BCE_ROWS, 128), lambda i: (i, 0))
    sds = jax.ShapeDtypeStruct((_EROWS_PAD, 128), jnp.float32)
    sdsi = jax.ShapeDtypeStruct((_EROWS_PAD, 128), jnp.int32)
    return pl.pallas_call(
        _bond_body,
        grid=(nblk,),
        in_specs=pspecs + [rspec, rspec],
        out_specs=[rspec, rspec],
        out_shape=[sds, sdsi],
    )(pb_tp, pb_tp, pb_tp, pb_tp, pb_tp, tb2, idxr)


def _zero_vmem(ref, nwords):
    z = jnp.zeros((16,), jnp.float32)

    def body(i, _):
        ref[pl.ds(i * 16, 16)] = z
        return 0

    lax.fori_loop(0, nwords // 16, body, 0)


def _bond_scatter_body(ce_hbm, idx_hbm, s0_hbm, s1_hbm, c0_hbm, c1_hbm,
                       idx_v, val_v, ones_v, zero_v, out_v2,
                       acc_s, cnt_s, sem):
    ci = lax.axis_index("c")
    si = lax.axis_index("s")
    wid = si * _NC + ci

    one = jnp.ones((16,), jnp.float32)
    for v in range(8):
        ones_v[pl.ds(v * 16, 16)] = one
    _zero_vmem(zero_v, _TILE_N)
    pltpu.sync_copy(zero_v, acc_s.at[pl.ds(si * _TILE_N, _TILE_N)])
    pltpu.sync_copy(zero_v, cnt_s.at[pl.ds(si * _TILE_N, _TILE_N)])
    plsc.subcore_barrier()

    def blk_body(bi, _):
        base = wid * _BROWS_W + bi * _BBLK
        pltpu.sync_copy(idx_hbm.at[pl.ds(base, _BBLK)], idx_v)
        pltpu.sync_copy(ce_hbm.at[pl.ds(base, _BBLK)], val_v)
        for g in range(0, _BBLK, 14):
            descs = []
            for j in range(g, g + 14):
                descs.append(pltpu.async_copy(
                    val_v.at[j], acc_s.at[idx_v.at[j]], sem, add=True))
                descs.append(pltpu.async_copy(
                    ones_v, cnt_s.at[idx_v.at[j]], sem, add=True))
            for d in descs:
                d.wait()
        return 0

    lax.fori_loop(0, _BNBLK, blk_body, 0)
    plsc.subcore_barrier()

    base = si * _TILE_N
    outs = [(s0_hbm, c0_hbm), (s1_hbm, c1_hbm)]
    for c in range(_NC):
        @pl.when(ci == c)
        def _():
            for src, dst in ((acc_s, outs[c][0]), (cnt_s, outs[c][1])):
                descs = [pltpu.async_copy(
                    src.at[pl.ds(base + r * 128, 128)], out_v2.at[r], sem)
                    for r in range(64)]
                for d in descs:
                    d.wait()
                pltpu.sync_copy(out_v2, dst.at[si])


def _bond_scatter(ce2, idx2):
    mesh = plsc.VectorSubcoreMesh(core_axis_name="c", subcore_axis_name="s",
                                  num_cores=_NC, num_subcores=_NS)
    sds = jax.ShapeDtypeStruct((_NS, 64, 128), jnp.float32)
    f = pl.kernel(
        _bond_scatter_body,
        out_type=[sds, sds, sds, sds],
        mesh=mesh,
        scratch_types=[
            pltpu.VMEM((_BBLK, 128), jnp.int32),
            pltpu.VMEM((_BBLK, 128), jnp.float32),
            pltpu.VMEM((128,), jnp.float32),
            pltpu.VMEM((_TILE_N,), jnp.float32),
            pltpu.VMEM((64, 128), jnp.float32),
            pltpu.VMEM_SHARED((_NP,), jnp.float32),
            pltpu.VMEM_SHARED((_NP,), jnp.float32),
            pltpu.SemaphoreType.DMA,
        ],
    )
    return f(ce2, idx2)


_SEG_L = 8192
_SEG_NBLK = 13


def _onehot_hi(bat):
    return jnp.where(
        lax.broadcasted_iota(jnp.int32, (128, _SEG_L), 0) == bat // 16,
        1.0, 0.0).astype(jnp.bfloat16)


def _lomask(bat):
    return lax.broadcasted_iota(jnp.int32, (16, _SEG_L), 0) == bat % 16


def _seg1_body(bat_ref, regr_ref, ace_ref, cce_ref, out_ref, macc):
    i = pl.program_id(0)
    bat = bat_ref[...]
    valid = (lax.broadcasted_iota(jnp.int32, (1, _SEG_L), 1) + i * _SEG_L) < _N
    onesrow = jnp.where(valid, 1.0, 0.0)
    lom = _lomask(bat)
    vals = [regr_ref[0], ace_ref[0], cce_ref[0], onesrow]
    vm = jnp.concatenate(
        [jnp.where(lom, v, 0.0) for v in vals], axis=0).astype(jnp.bfloat16)
    contrib = lax.dot_general(_onehot_hi(bat), vm, (((1,), (1,)), ((), ())),
                              preferred_element_type=jnp.float32)

    @pl.when(i == 0)
    def _():
        macc[...] = contrib

    @pl.when(i > 0)
    def _():
        macc[...] = macc[...] + contrib

    @pl.when(i == _SEG_NBLK - 1)
    def _():
        out_ref[...] = macc[...]


def _seg1(bat2, regr3, ace3, cce3):
    spec = pl.BlockSpec((1, 1, _SEG_L), lambda i: (i, 0, 0))
    bspec = pl.BlockSpec((1, _SEG_L), lambda i: (0, i))
    return pl.pallas_call(
        _seg1_body,
        grid=(_SEG_NBLK,),
        in_specs=[bspec] + [spec] * 3,
        out_specs=pl.BlockSpec((128, 64), lambda i: (0, 0)),
        out_shape=jax.ShapeDtypeStruct((128, 64), jnp.float32),
        scratch_shapes=[pltpu.VMEM((128, 64), jnp.float32)],
    )(bat2, regr3, ace3, cce3)


def _seg2_body(bat_ref, s0_ref, s1_ref, c0_ref, c1_ref, m1_ref, w_ref,
               out_ref, macc):
    i = pl.program_id(0)
    bat = bat_ref[...]
    flat = (lax.broadcasted_iota(jnp.int32, (64, 128), 0) * 128
            + lax.broadcasted_iota(jnp.int32, (64, 128), 1) + i * _SEG_L)
    sm = s0_ref[0] + s1_ref[0]
    cm = c0_ref[0] + c1_ref[0]
    b2 = jnp.where(flat < _N, (0.5 * sm) / jnp.maximum(cm, 1.0), 0.0)
    b = b2.reshape(1, _SEG_L)
    lom = _lomask(bat)
    vm = jnp.where(lom, b, 0.0).astype(jnp.bfloat16)
    contrib = lax.dot_general(_onehot_hi(bat), vm, (((1,), (1,)), ((), ())),
                              preferred_element_type=jnp.float32)

    @pl.when(i == 0)
    def _():
        macc[...] = contrib

    @pl.when(i > 0)
    def _():
        macc[...] = macc[...] + contrib

    @pl.when(i == _SEG_NBLK - 1)
    def _():
        m1 = m1_ref[...]
        cnt = jnp.maximum(m1[:, 48:64], 1.0)
        w = w_ref[...]
        cols = []
        for k in range(3):
            mk = m1[:, k * 16:(k + 1) * 16] / cnt
            mk = jnp.where(jnp.isnan(mk), 0.0, mk * w)
            cols.append(jnp.sum(mk, axis=1, keepdims=True))
        mb = macc[...] / cnt
        mb = jnp.where(jnp.isnan(mb), 0.0, mb * w)
        cols.append(jnp.sum(mb, axis=1, keepdims=True))
        c4 = jnp.concatenate([cols[0], cols[1], cols[2], cols[3]], axis=1)
        out_ref[...] = jnp.sum(c4, axis=0, keepdims=True)


def _seg2(bat2, s03, s13, c03, c13, m1, w128):
    spec = pl.BlockSpec((1, 64, 128), lambda i: (i, 0, 0))
    bspec = pl.BlockSpec((1, _SEG_L), lambda i: (0, i))
    return pl.pallas_call(
        _seg2_body,
        grid=(_SEG_NBLK,),
        in_specs=[bspec] + [spec] * 4
        + [pl.BlockSpec((128, 64), lambda i: (0, 0)),
           pl.BlockSpec((128, 16), lambda i: (0, 0))],
        out_specs=pl.BlockSpec((1, 4), lambda i: (0, 0)),
        out_shape=jax.ShapeDtypeStruct((1, 4), jnp.float32),
        scratch_shapes=[pltpu.VMEM((128, 16), jnp.float32)],
    )(bat2, s03, s13, c03, c13, m1, w128)


@jax.jit
def kernel(pred_coords, true_coords, pred_atoms, true_atoms, pred_charges,
           true_charges, pred_bonds, true_bonds, batch,
           bond_aggregation_index, weights):
    idxr = bond_aggregation_index.reshape(_EROWS, 128)
    tb2 = true_bonds.reshape(_EROWS, 128)
    pb_tp = jnp.pad(pred_bonds, ((0, _EP - _E), (0, 0))).T.reshape(
        _BC, _EROWS_PAD, 128)
    ce, idx2 = _bond_ce(pb_tp, tb2, idxr)

    regr, ace, cce = _atom_values(
        pred_coords.T, true_coords.T, pred_atoms.T,
        true_atoms.reshape(1, _N), pred_charges.T,
        true_charges.reshape(1, _N))
    bat2 = batch.reshape(1, _N)
    m1 = _seg1(bat2, regr, ace, cce)

    s0, s1, c0, c1 = _bond_scatter(ce, idx2)
    out = _seg2(bat2, s0, s1, c0, c1, m1, weights.reshape(128, 16))
    return out.reshape(4)


# speedup vs baseline: 1.0081x; 1.0004x over previous
import jax
import jax.numpy as jnp
from jax import lax
from jax.experimental import pallas as pl
from jax.experimental.pallas import tpu as pltpu
from jax.experimental.pallas import tpu_sc as plsc

_N = 100000
_E = 1600000
_B = 2048
_AC = 16
_CC = 6
_BC = 5

_LAA = 8192
_LAB = 6400
_NP = 131072
_NROWS = _NP // 128
_EROWS_PAD = 12544
_EP = _EROWS_PAD * 128

_NC = 2
_NS = 16
_NW = _NC * _NS

_BROWS_W = _EROWS_PAD // _NW
_BBLK = 56
_BNBLK = _BROWS_W // _BBLK
_AROWS_W = _NROWS // _NW
_TILE_N = _NP // _NS


def _atom_body(pc_ref, tc_ref, pa_ref, ta_ref, pch_ref, tch_ref,
               regr_ref, ace_ref, cce_ref):
    blk = pl.program_id(0)
    lanes = lax.broadcasted_iota(jnp.int32, (1, _LAA), 1) + blk * _LAA
    mask = lanes < _N

    d = pc_ref[...] - tc_ref[...]
    regr = jnp.sum(d * d, axis=0, keepdims=True) * (1.0 / 3.0)
    regr_ref[...] = jnp.where(mask, regr, 0.0)[None]

    pa = pa_ref[...]
    ta = ta_ref[...]
    m = jnp.max(pa, axis=0, keepdims=True)
    lse = jnp.log(jnp.sum(jnp.exp(pa - m), axis=0, keepdims=True)) + m
    onehot = lax.broadcasted_iota(jnp.int32, (_AC, _LAA), 0) == ta
    tgt = jnp.sum(jnp.where(onehot, pa, 0.0), axis=0, keepdims=True)
    ace_ref[...] = jnp.where(mask, lse - tgt, 0.0)[None]

    pch = pch_ref[...]
    tch = tch_ref[...]
    m2 = jnp.max(pch, axis=0, keepdims=True)
    lse2 = jnp.log(jnp.sum(jnp.exp(pch - m2), axis=0, keepdims=True)) + m2
    onehot2 = lax.broadcasted_iota(jnp.int32, (_CC, _LAA), 0) == tch
    tgt2 = jnp.sum(jnp.where(onehot2, pch, 0.0), axis=0, keepdims=True)
    cce_ref[...] = jnp.where(mask, lse2 - tgt2, 0.0)[None]


_ANBLK = 13


def _atom_values(pc_t, tc_t, pa_t, ta2, pch_t, tch2):
    out_sds = jax.ShapeDtypeStruct((_ANBLK, 1, _LAA), jnp.float32)
    lane = lambda r: pl.BlockSpec((r, _LAA), lambda i: (0, i))
    return pl.pallas_call(
        _atom_body,
        grid=(_ANBLK,),
        in_specs=[lane(3), lane(3), lane(_AC), lane(1), lane(_CC), lane(1)],
        out_specs=[pl.BlockSpec((1, 1, _LAA), lambda i: (i, 0, 0))] * 3,
        out_shape=[out_sds, out_sds, out_sds],
    )(pc_t, tc_t, pa_t, ta2, pch_t, tch2)


_BCE_ROWS = 896


_EROWS = _E // 128


def _bond_body(p0_ref, p1_ref, p2_ref, p3_ref, p4_ref, tb_ref, idx_ref,
               ce_ref, idx2_ref):
    i = pl.program_id(0)
    ps = [p0_ref[0], p1_ref[0], p2_ref[0], p3_ref[0], p4_ref[0]]
    tb = tb_ref[...]
    m = jnp.maximum(jnp.maximum(jnp.maximum(ps[0], ps[1]),
                                jnp.maximum(ps[2], ps[3])), ps[4])
    ssum = jnp.exp(ps[0] - m)
    for k in range(1, _BC):
        ssum = ssum + jnp.exp(ps[k] - m)
    lse = jnp.log(ssum) + m
    tgt = jnp.where(tb == 0, ps[0], 0.0)
    for k in range(1, _BC):
        tgt = jnp.where(tb == k, ps[k], tgt)
    ce_ref[...] = lse - tgt

    rows = lax.broadcasted_iota(jnp.int32, (_BCE_ROWS, 128), 0) + i * _BCE_ROWS
    idx2_ref[...] = jnp.where(rows < _EROWS, idx_ref[...], _N)


def _bond_ce(pb_tp, tb2, idxr):
    nblk = _EROWS_PAD // _BCE_ROWS
    pspecs = [pl.BlockSpec((1, _BCE_ROWS, 128),
                           lambda i, k=k: (k, i, 0)) for k in range(_BC)]
    rspec = pl.BlockSpec((_BCE_ROWS, 128), lambda i: (i, 0))
    sds = jax.ShapeDtypeStruct((_EROWS_PAD, 128), jnp.float32)
    sdsi = jax.ShapeDtypeStruct((_EROWS_PAD, 128), jnp.int32)
    return pl.pallas_call(
        _bond_body,
        grid=(nblk,),
        in_specs=pspecs + [rspec, rspec],
        out_specs=[rspec, rspec],
        out_shape=[sds, sdsi],
    )(pb_tp, pb_tp, pb_tp, pb_tp, pb_tp, tb2, idxr)


def _zero_vmem(ref, nwords):
    z = jnp.zeros((16,), jnp.float32)

    def body(i, _):
        ref[pl.ds(i * 16, 16)] = z
        return 0

    lax.fori_loop(0, nwords // 16, body, 0)


def _bond_scatter_body(ce_hbm, idx_hbm, s0_hbm, s1_hbm, c0_hbm, c1_hbm,
                       idx_v, val_v, ones_v, zero_v, out_v2,
                       acc_s, cnt_s, sem):
    ci = lax.axis_index("c")
    si = lax.axis_index("s")
    wid = si * _NC + ci

    one = jnp.ones((16,), jnp.float32)
    for v in range(8):
        ones_v[pl.ds(v * 16, 16)] = one
    _zero_vmem(zero_v, _TILE_N)
    pltpu.sync_copy(zero_v, acc_s.at[pl.ds(si * _TILE_N, _TILE_N)])
    pltpu.sync_copy(zero_v, cnt_s.at[pl.ds(si * _TILE_N, _TILE_N)])
    plsc.subcore_barrier()

    def blk_body(bi, _):
        base = wid * _BROWS_W + bi * _BBLK
        pltpu.sync_copy(idx_hbm.at[pl.ds(base, _BBLK)], idx_v)
        pltpu.sync_copy(ce_hbm.at[pl.ds(base, _BBLK)], val_v)
        for g in range(0, _BBLK, 14):
            descs = []
            for j in range(g, g + 14):
                descs.append(pltpu.async_copy(
                    val_v.at[j], acc_s.at[idx_v.at[j]], sem, add=True))
                descs.append(pltpu.async_copy(
                    ones_v, cnt_s.at[idx_v.at[j]], sem, add=True))
            for d in descs:
                d.wait()
        return 0

    lax.fori_loop(0, _BNBLK, blk_body, 0)
    plsc.subcore_barrier()

    base = si * _TILE_N
    outs = [(s0_hbm, c0_hbm), (s1_hbm, c1_hbm)]
    for c in range(_NC):
        @pl.when(ci == c)
        def _():
            for src, dst in ((acc_s, outs[c][0]), (cnt_s, outs[c][1])):
                descs = [pltpu.async_copy(
                    src.at[pl.ds(base + r * 128, 128)], out_v2.at[r], sem)
                    for r in range(64)]
                for d in descs:
                    d.wait()
                pltpu.sync_copy(out_v2, dst.at[si])


def _bond_scatter(ce2, idx2):
    mesh = plsc.VectorSubcoreMesh(core_axis_name="c", subcore_axis_name="s",
                                  num_cores=_NC, num_subcores=_NS)
    sds = jax.ShapeDtypeStruct((_NS, 64, 128), jnp.float32)
    f = pl.kernel(
        _bond_scatter_body,
        out_type=[sds, sds, sds, sds],
        mesh=mesh,
        scratch_types=[
            pltpu.VMEM((_BBLK, 128), jnp.int32),
            pltpu.VMEM((_BBLK, 128), jnp.float32),
            pltpu.VMEM((128,), jnp.float32),
            pltpu.VMEM((_TILE_N,), jnp.float32),
            pltpu.VMEM((64, 128), jnp.float32),
            pltpu.VMEM_SHARED((_NP,), jnp.float32),
            pltpu.VMEM_SHARED((_NP,), jnp.float32),
            pltpu.SemaphoreType.DMA,
        ],
    )
    return f(ce2, idx2)


_SEG_L = 8192
_SEG_NBLK = 13


def _onehot_hi(bat):
    return jnp.where(
        lax.broadcasted_iota(jnp.int32, (128, _SEG_L), 0) == bat // 16,
        1.0, 0.0).astype(jnp.bfloat16)


def _lomask(bat):
    return lax.broadcasted_iota(jnp.int32, (16, _SEG_L), 0) == bat % 16


def _seg1_body(bat_ref, regr_ref, ace_ref, cce_ref, out_ref, macc):
    i = pl.program_id(0)
    bat = bat_ref[...]
    valid = (lax.broadcasted_iota(jnp.int32, (1, _SEG_L), 1) + i * _SEG_L) < _N
    onesrow = jnp.where(valid, 1.0, 0.0)
    lom = _lomask(bat)
    vals = [regr_ref[0], ace_ref[0], cce_ref[0], onesrow]
    vm = jnp.concatenate(
        [jnp.where(lom, v, 0.0) for v in vals], axis=0).astype(jnp.bfloat16)
    contrib = lax.dot_general(_onehot_hi(bat), vm, (((1,), (1,)), ((), ())),
                              preferred_element_type=jnp.float32)

    @pl.when(i == 0)
    def _():
        macc[...] = contrib

    @pl.when(i > 0)
    def _():
        macc[...] = macc[...] + contrib

    @pl.when(i == _SEG_NBLK - 1)
    def _():
        out_ref[...] = macc[...]


def _seg1(bat2, regr3, ace3, cce3):
    spec = pl.BlockSpec((1, 1, _SEG_L), lambda i: (i, 0, 0))
    bspec = pl.BlockSpec((1, _SEG_L), lambda i: (0, i))
    return pl.pallas_call(
        _seg1_body,
        grid=(_SEG_NBLK,),
        in_specs=[bspec] + [spec] * 3,
        out_specs=pl.BlockSpec((128, 64), lambda i: (0, 0)),
        out_shape=jax.ShapeDtypeStruct((128, 64), jnp.float32),
        scratch_shapes=[pltpu.VMEM((128, 64), jnp.float32)],
    )(bat2, regr3, ace3, cce3)


def _seg2_body(bat_ref, s0_ref, s1_ref, c0_ref, c1_ref, m1_ref, w_ref,
               out_ref, macc):
    i = pl.program_id(0)
    bat = bat_ref[...]
    flat = (lax.broadcasted_iota(jnp.int32, (64, 128), 0) * 128
            + lax.broadcasted_iota(jnp.int32, (64, 128), 1) + i * _SEG_L)
    sm = s0_ref[0] + s1_ref[0]
    cm = c0_ref[0] + c1_ref[0]
    b2 = jnp.where(flat < _N, (0.5 * sm) / jnp.maximum(cm, 1.0), 0.0)
    b = b2.reshape(1, _SEG_L)
    lom = _lomask(bat)
    vm = jnp.where(lom, b, 0.0).astype(jnp.bfloat16)
    contrib = lax.dot_general(_onehot_hi(bat), vm, (((1,), (1,)), ((), ())),
                              preferred_element_type=jnp.float32)

    @pl.when(i == 0)
    def _():
        macc[...] = contrib

    @pl.when(i > 0)
    def _():
        macc[...] = macc[...] + contrib

    @pl.when(i == _SEG_NBLK - 1)
    def _():
        m1 = m1_ref[...]
        cnt = jnp.maximum(m1[:, 48:64], 1.0)
        w = w_ref[...]
        cols = []
        for k in range(3):
            mk = m1[:, k * 16:(k + 1) * 16] / cnt
            mk = jnp.where(jnp.isnan(mk), 0.0, mk * w)
            cols.append(jnp.sum(mk, axis=1, keepdims=True))
        mb = macc[...] / cnt
        mb = jnp.where(jnp.isnan(mb), 0.0, mb * w)
        cols.append(jnp.sum(mb, axis=1, keepdims=True))
        c4 = jnp.concatenate([cols[0], cols[1], cols[2], cols[3]], axis=1)
        out_ref[...] = jnp.sum(c4, axis=0, keepdims=True)


def _seg2(bat2, s03, s13, c03, c13, m1, w128):
    spec = pl.BlockSpec((1, 64, 128), lambda i: (i, 0, 0))
    bspec = pl.BlockSpec((1, _SEG_L), lambda i: (0, i))
    return pl.pallas_call(
        _seg2_body,
        grid=(_SEG_NBLK,),
        in_specs=[bspec] + [spec] * 4
        + [pl.BlockSpec((128, 64), lambda i: (0, 0)),
           pl.BlockSpec((128, 16), lambda i: (0, 0))],
        out_specs=pl.BlockSpec((1, 4), lambda i: (0, 0)),
        out_shape=jax.ShapeDtypeStruct((1, 4), jnp.float32),
        scratch_shapes=[pltpu.VMEM((128, 16), jnp.float32)],
    )(bat2, s03, s13, c03, c13, m1, w128)


@jax.jit
def kernel(pred_coords, true_coords, pred_atoms, true_atoms, pred_charges,
           true_charges, pred_bonds, true_bonds, batch,
           bond_aggregation_index, weights):
    idxr = bond_aggregation_index.reshape(_EROWS, 128)
    tb2 = true_bonds.reshape(_EROWS, 128)
    pb_tp = jnp.pad(pred_bonds, ((0, _EP - _E), (0, 0))).T.reshape(
        _BC, _EROWS_PAD, 128)
    ce, idx2 = _bond_ce(pb_tp, tb2, idxr)

    regr, ace, cce = _atom_values(
        pred_coords.T, true_coords.T, pred_atoms.T,
        true_atoms.reshape(1, _N), pred_charges.T,
        true_charges.reshape(1, _N))
    bat2 = batch.reshape(1, _N)
    m1 = _seg1(bat2, regr, ace, cce)

    s0, s1, c0, c1 = _bond_scatter(ce, idx2)
    out = _seg2(bat2, s0, s1, c0, c1, m1, weights.reshape(128, 16))
    return out.reshape(4)
